# Initial kernel scaffold; baseline (speedup 1.0000x reference)
import jax
import jax.numpy as jnp
from jax.experimental import pallas as pl


def kernel(h_src, h_dst, edge_index, x_dst_orig, edge_dt, params):
    raise NotImplementedError("write your pallas kernel here")


# speedup vs baseline: 18.3197x; 18.3197x over previous
import functools
import math

import jax
import jax.numpy as jnp
from jax import lax
from jax.experimental import pallas as pl
from jax.experimental.pallas import tpu as pltpu
from jax.experimental.pallas import tpu_sc as plsc

NS = 10000
ND = 10000
E = 160000
D = 256
NH = 8
DH = D // NH
MAXB = 512
NB = 32

NCORES = 2
NSUB = 16
NTILES = NCORES * NSUB
EPT = 5120
EPAD = EPT * NTILES
NCHUNK = 4
CD = 64
ACCW = 128
EXW = 16
NDPAD = 10240
ROWS_PT = NDPAD // NSUB

_mesh = plsc.VectorSubcoreMesh(core_axis_name="c", subcore_axis_name="s")

f32 = jnp.float32
i32 = jnp.int32


DW = 128
DROWS = 10240
DRPT = DROWS // NSUB
DGB = 128


@functools.partial(
    pl.kernel,
    out_type=jax.ShapeDtypeStruct((NCORES * 2 * DROWS, DW), f32),
    mesh=_mesh,
    scratch_types=[pltpu.VMEM((DGB,), i32),
                   pltpu.VMEM((DGB, DW), f32),
                   pltpu.VMEM((DGB, DW), f32),
                   pltpu.VMEM_SHARED((DROWS, DW), f32)],
)
def _deg_kernel(u_hbm, v_hbm, out_hbm, si_v, ones_v, zb, acc):
    core = lax.axis_index("c")
    sub = lax.axis_index("s")
    wid = sub * NCORES + core
    base = wid * EPT
    zf = jnp.zeros((16,), f32)
    of = jnp.ones((16,), f32)
    for r in range(DGB):
        for kk in range(DW // 16):
            zb[r, pl.ds(kk * 16, 16)] = zf
            ones_v[r, pl.ds(kk * 16, 16)] = of

    arow = sub * DRPT

    for p, idx_hbm in ((0, u_hbm), (1, v_hbm)):
        def zacc(b, _):
            pltpu.sync_copy(zb, acc.at[pl.ds(arow + b * DGB, DGB)])
            return _
        lax.fori_loop(0, DRPT // DGB, zacc, None)
        plsc.subcore_barrier()

        def group(g, _):
            eb = base + g * DGB
            pltpu.sync_copy(idx_hbm.at[pl.ds(eb, DGB)], si_v)
            pltpu.sync_copy(ones_v, acc.at[si_v], add=True)
            return _
        lax.fori_loop(0, EPT // DGB, group, None)
        plsc.subcore_barrier()

        obase = (core * 2 + p) * DROWS

        def drain(b, _):
            r = arow + b * DGB
            pltpu.sync_copy(acc.at[pl.ds(r, DGB)],
                            out_hbm.at[pl.ds(obase + r, DGB)])
            return _
        lax.fori_loop(0, DRPT // DGB, drain, None)
        plsc.subcore_barrier()


_B = 1000


def _ln_block(x, g, b):
    mu = jnp.mean(x, axis=-1, keepdims=True)
    var = jnp.mean((x - mu) ** 2, axis=-1, keepdims=True)
    return (x - mu) / jnp.sqrt(var + 1e-5) * g + b


def _prep_body(hs_ref, hd_ref, x_ref, dgu_ref, dgv_ref, embo_ref, embi_ref,
               cin0_ref, cout0_ref, lnqg_ref, lnqb_ref, lnkg_ref, lnkb_ref,
               wq_ref, wk_ref, wv_ref, w0_ref, w1_ref, pb_ref,
               q_ref, k_ref, v_ref, p2_ref):
    log16 = 1.0 / math.log(1.6)

    def bucket_onehot(deg_col):
        bb = jnp.floor(jnp.log1p(deg_col) * log16).astype(i32)
        bb = jnp.clip(bb, 0, NB - 1)
        return (bb ==
                lax.broadcasted_iota(i32, (_B, NB), 1)).astype(f32)

    degu = dgu_ref[...]
    cent_s = jnp.dot(bucket_onehot(degu), embo_ref[...],
                     preferred_element_type=f32) + cin0_ref[...]
    hs = hs_ref[...] + cent_s
    hk = _ln_block(hs, lnkg_ref[...], lnkb_ref[...])
    k_ref[...] = jnp.dot(hk, wk_ref[...], preferred_element_type=f32)
    v_ref[...] = jnp.dot(hk, wv_ref[...], preferred_element_type=f32)
    degv = dgv_ref[...]
    cent_d = jnp.dot(bucket_onehot(degv), embi_ref[...],
                     preferred_element_type=f32) + cout0_ref[...]
    hd = hd_ref[...] + cent_d
    hq = _ln_block(hd, lnqg_ref[...], lnqb_ref[...])
    q_ref[...] = jnp.dot(hq, wq_ref[...], preferred_element_type=f32)
    p2_ref[...] = (jnp.dot(hd, w0_ref[...], preferred_element_type=f32)
                   + jnp.dot(x_ref[...], w1_ref[...],
                             preferred_element_type=f32)
                   + pb_ref[...])


def _prep_call(hs, hd, x, degu, degv, embo, embi, cin0, cout0,
               lnqg, lnqb, lnkg, lnkb, wq, wk, wv, w0, w1, pb):
    n_blk = NS // _B
    row = lambda i: (i, 0)
    full = lambda i: (0, 0)
    return pl.pallas_call(
        _prep_body,
        grid=(n_blk,),
        in_specs=[
            pl.BlockSpec((_B, D), row), pl.BlockSpec((_B, D), row),
            pl.BlockSpec((_B, D), row),
            pl.BlockSpec((_B, 1), row),
            pl.BlockSpec((_B, 1), row),
            pl.BlockSpec((NB, D), full), pl.BlockSpec((NB, D), full),
            pl.BlockSpec((1, D), full), pl.BlockSpec((1, D), full),
            pl.BlockSpec((1, D), full), pl.BlockSpec((1, D), full),
            pl.BlockSpec((1, D), full), pl.BlockSpec((1, D), full),
            pl.BlockSpec((D, D), full), pl.BlockSpec((D, D), full),
            pl.BlockSpec((D, D), full), pl.BlockSpec((D, D), full),
            pl.BlockSpec((D, D), full), pl.BlockSpec((1, D), full),
        ],
        out_specs=[
            pl.BlockSpec((_B, D), row), pl.BlockSpec((_B, D), row),
            pl.BlockSpec((_B, D), row), pl.BlockSpec((_B, D), row),
        ],
        out_shape=[
            jax.ShapeDtypeStruct((ND, D), f32),
            jax.ShapeDtypeStruct((NS, D), f32),
            jax.ShapeDtypeStruct((NS, D), f32),
            jax.ShapeDtypeStruct((ND, D), f32),
        ],
    )(hs, hd, x, degu, degv, embo, embi, cin0, cout0,
      lnqg, lnqb, lnkg, lnkb, wq, wk, wv, w0, w1, pb)


GGB = 128


@functools.partial(
    pl.kernel,
    out_type=[jax.ShapeDtypeStruct((EPAD, D), f32),
              jax.ShapeDtypeStruct((EPAD, D), f32),
              jax.ShapeDtypeStruct((EPAD, D), f32)],
    mesh=_mesh,
    scratch_types=[
        pltpu.VMEM((GGB,), i32),
        pltpu.VMEM((GGB,), i32),
        pltpu.VMEM((GGB, D), f32),
        pltpu.VMEM((GGB, D), f32),
        pltpu.VMEM((GGB, D), f32),
        pltpu.SemaphoreType.DMA,
        pltpu.SemaphoreType.DMA,
        pltpu.SemaphoreType.DMA,
    ],
)
def _gather_kernel(q_hbm, k_hbm, v_hbm, ug_hbm, vg_hbm,
                   qe_hbm, ke_hbm, ve_hbm,
                   ui_v, vi_v, qb, kb, vb, s1, s2, s3):
    core = lax.axis_index("c")
    sub = lax.axis_index("s")
    wid = sub * NCORES + core
    base = wid * EPT

    def group(g, _):
        eb = base + g * GGB
        pltpu.sync_copy(ug_hbm.at[pl.ds(eb, GGB)], ui_v)
        pltpu.sync_copy(vg_hbm.at[pl.ds(eb, GGB)], vi_v)
        cq = pltpu.async_copy(q_hbm.at[vi_v], qb, s1)
        ck = pltpu.async_copy(k_hbm.at[ui_v], kb, s2)
        cv = pltpu.async_copy(v_hbm.at[ui_v], vb, s3)
        cq.wait()
        ck.wait()
        cv.wait()
        pltpu.sync_copy(qb, qe_hbm.at[pl.ds(eb, GGB)])
        pltpu.sync_copy(kb, ke_hbm.at[pl.ds(eb, GGB)])
        pltpu.sync_copy(vb, ve_hbm.at[pl.ds(eb, GGB)])
        return _
    lax.fori_loop(0, EPT // GGB, group, None)


EB = 2048


def _edge_tc_body(qe_ref, ke_ref, ve_ref, dt_ref, ivf_ref,
                  hmap_ref, hmapt_ref, w_ref, ex_ref):
    prods = qe_ref[...] * ke_ref[...]
    logits = jnp.dot(prods, hmap_ref[...], preferred_element_type=f32)
    logits = logits - jnp.maximum(dt_ref[...], 0.0) * ivf_ref[...]
    ex = jnp.exp(logits)
    exfull = jnp.dot(ex, hmapt_ref[...], preferred_element_type=f32)
    ve = ve_ref[...]
    wv = exfull * ve
    for c in range(NCHUNK):
        w_ref[c] = jnp.concatenate(
            [wv[:, c * CD:(c + 1) * CD], ve[:, c * CD:(c + 1) * CD]], axis=1)
    ex_ref[...] = jnp.concatenate([ex, jnp.zeros((EB, ACCW - NH), f32)],
                                  axis=1)


def _edge_tc_call(qe, ke, ve, dt, ivf, hmap, hmapt):
    n_blk = EPAD // EB
    row = lambda i: (i, 0)
    full = lambda i: (0, 0)
    return pl.pallas_call(
        _edge_tc_body,
        grid=(n_blk,),
        in_specs=[
            pl.BlockSpec((EB, D), row), pl.BlockSpec((EB, D), row),
            pl.BlockSpec((EB, D), row), pl.BlockSpec((EB, 1), row),
            pl.BlockSpec((1, NH), full),
            pl.BlockSpec((D, NH), full), pl.BlockSpec((NH, D), full),
        ],
        out_specs=[
            pl.BlockSpec((NCHUNK, EB, ACCW), lambda i: (0, i, 0)),
            pl.BlockSpec((EB, ACCW), row),
        ],
        out_shape=[
            jax.ShapeDtypeStruct((NCHUNK, EPAD, ACCW), f32),
            jax.ShapeDtypeStruct((EPAD, ACCW), f32),
        ],
    )(qe, ke, ve, dt, ivf, hmap, hmapt)


SGB = 128


NPASS = NCHUNK + 1


@functools.partial(
    pl.kernel,
    out_type=jax.ShapeDtypeStruct((NCORES * NPASS * NDPAD, ACCW), f32),
    mesh=_mesh,
    scratch_types=[
        pltpu.VMEM((SGB,), i32),
        pltpu.VMEM((SGB, ACCW), f32),
        pltpu.VMEM((8, ACCW), f32),
        pltpu.VMEM_SHARED((NDPAD, ACCW), f32),
    ],
)
def _scatter_kernel(w_hbm, vs_hbm, acc_hbm, si_v, wb, zb, acc):
    core = lax.axis_index("c")
    sub = lax.axis_index("s")
    wid = sub * NCORES + core
    base = wid * EPT
    zf = jnp.zeros((16,), f32)
    for r in range(8):
        for k in range(ACCW // 16):
            zb[r, pl.ds(k * 16, 16)] = zf

    arow = sub * ROWS_PT

    def pass_body(c, _):
        def zacc(b, _):
            pltpu.sync_copy(zb, acc.at[pl.ds(arow + b * 8, 8)])
            return _
        lax.fori_loop(0, ROWS_PT // 8, zacc, None)
        plsc.subcore_barrier()

        def group(g, _):
            eb = base + g * SGB
            pltpu.sync_copy(vs_hbm.at[pl.ds(eb, SGB)], si_v)
            pltpu.sync_copy(w_hbm.at[pl.ds(c * EPAD + eb, SGB)], wb)
            pltpu.sync_copy(wb, acc.at[si_v], add=True)
            return _
        lax.fori_loop(0, EPT // SGB, group, None)
        plsc.subcore_barrier()

        obase = (core * NPASS + c) * NDPAD

        def dr(b, _):
            r = arow + b * 80

            @pl.when(r < ND)
            def _d():
                pltpu.sync_copy(acc.at[pl.ds(r, 80)],
                                acc_hbm.at[pl.ds(obase + r, 80)])
            return _
        lax.fori_loop(0, ROWS_PT // 80, dr, None)
        plsc.subcore_barrier()
        return _
    lax.fori_loop(0, NPASS, pass_body, None)


def _finish_body(hacc_ref, exa_ref, dgv_ref, p2_ref, mix_ref, wo_ref,
                 wl_ref, wlb_ref, flng_ref, flnb_ref, fw1_ref, fb1_ref,
                 fw2_ref, fb2_ref, out_ref):
    m = mix_ref[0, 0]
    deg = dgv_ref[...]
    invdeg = 1.0 / jnp.maximum(deg, 1.0)
    exs = exa_ref[...]
    parts = []
    for c in range(NCHUNK):
        hp = hacc_ref[c]
        for hh in range(2):
            a = hp[:, hh * 32:(hh + 1) * 32]
            ss = exs[:, 2 * c + hh:2 * c + hh + 1]
            vs = hp[:, 64 + hh * 32:64 + (hh + 1) * 32]
            attn = a / (ss + 1e-16)
            mean = vs * invdeg
            parts.append(m * attn + (1.0 - m) * mean)
    Hc = jnp.concatenate(parts, axis=1)
    o1 = jnp.dot(Hc, wo_ref[...], preferred_element_type=f32)
    out = (jnp.dot(o1, wl_ref[...], preferred_element_type=f32)
           + wlb_ref[...] + p2_ref[...])
    fx = _ln_block(out, flng_ref[...], flnb_ref[...])
    fx = jnp.dot(fx, fw1_ref[...], preferred_element_type=f32) + fb1_ref[...]
    fx = 0.5 * fx * (1.0 + lax.erf(fx * (1.0 / math.sqrt(2.0))))
    fx = jnp.dot(fx, fw2_ref[...], preferred_element_type=f32) + fb2_ref[...]
    out_ref[...] = out + fx


def _finish_call(hacc, exacc, deg, p2, mix, wo, wl, wlb, flng, flnb, fw1,
                 fb1, fw2, fb2):
    n_blk = ND // _B
    row = lambda i: (i, 0)
    full = lambda i: (0, 0)
    return pl.pallas_call(
        _finish_body,
        grid=(n_blk,),
        in_specs=[
            pl.BlockSpec((NCHUNK, _B, ACCW), lambda i: (0, i, 0)),
            pl.BlockSpec((_B, EXW), row),
            pl.BlockSpec((_B, 1), row),
            pl.BlockSpec((_B, D), row),
            pl.BlockSpec((1, 128), full),
            pl.BlockSpec((D, D), full), pl.BlockSpec((D, D), full),
            pl.BlockSpec((1, D), full),
            pl.BlockSpec((1, D), full), pl.BlockSpec((1, D), full),
            pl.BlockSpec((D, 4 * D), full), pl.BlockSpec((1, 4 * D), full),
            pl.BlockSpec((4 * D, D), full), pl.BlockSpec((1, D), full),
        ],
        out_specs=pl.BlockSpec((_B, D), row),
        out_shape=jax.ShapeDtypeStruct((ND, D), f32),
    )(hacc, exacc, deg, p2, mix, wo, wl, wlb, flng, flnb, fw1, fb1, fw2,
      fb2)


def kernel(h_src, h_dst, edge_index, x_dst_orig, edge_dt, params):
    p = params
    u = edge_index[0]
    v = edge_index[1]
    npad = EPAD - E
    u_g = jnp.concatenate([u, jnp.zeros((npad,), i32)])
    v_g = jnp.concatenate([v, jnp.zeros((npad,), i32)])
    v_s = jnp.concatenate([v, jnp.full((npad,), ND, i32)])
    dtp = jnp.concatenate([edge_dt, jnp.zeros((npad,), f32)]).reshape(EPAD, 1)
    u_deg = jnp.concatenate([u, jnp.full((npad,), NS, i32)])
    v_deg = jnp.concatenate([v, jnp.full((npad,), ND, i32)])

    deg = _deg_kernel(u_deg, v_deg)
    degu = deg[:NS, 0:1] + deg[2 * DROWS:2 * DROWS + NS, 0:1]
    degv = (deg[DROWS:DROWS + ND, 0:1]
            + deg[3 * DROWS:3 * DROWS + ND, 0:1])

    gate = jax.nn.sigmoid(p['gate'])
    tau = jnp.clip(jnp.exp(p['log_tau']), 0.5, 2.0)
    lam = jnp.maximum(p['lambda_dt'], 0.0)
    qscale = jnp.repeat(1.0 / (math.sqrt(DH) * tau), DH)
    wq = p['W_Q'] * qscale[None, :]
    w0 = (1.0 - gate) * p['w0_w']
    w1 = gate * p['w1_w']
    pb = ((1.0 - gate) * p['w0_b'] + gate * p['w1_b']).reshape(1, D)
    ac = p['alpha_c']
    embo = ac * p['deg_out_emb'][:NB]
    embi = ac * p['deg_in_emb'][:NB]
    cin0 = (ac * p['deg_in_emb'][0]).reshape(1, D)
    cout0 = (ac * p['deg_out_emb'][0]).reshape(1, D)
    ivf = (lam / tau).reshape(1, NH)
    mix = jnp.full((1, 128), jax.nn.sigmoid(p['mix_attn']), f32)
    hmap = (jnp.arange(D, dtype=i32)[:, None] // DH
            == jnp.arange(NH, dtype=i32)[None, :]).astype(f32)

    q, k, vv, p2 = _prep_call(
        h_src, h_dst, x_dst_orig, degu, degv, embo, embi, cin0, cout0,
        p['ln_q_g'].reshape(1, D), p['ln_q_b'].reshape(1, D),
        p['ln_kv_g'].reshape(1, D), p['ln_kv_b'].reshape(1, D),
        wq, p['W_K'], p['W_V'], w0, w1, pb)

    qe, ke, ve = _gather_kernel(q, k, vv, u_g, v_g)
    wrows, exrows = _edge_tc_call(qe, ke, ve, dtp, ivf, hmap, hmap.T)
    w_all = jnp.concatenate(
        [wrows.reshape(NCHUNK * EPAD, ACCW), exrows], axis=0)
    racc = _scatter_kernel(w_all, v_s).reshape(NCORES, NPASS, NDPAD, ACCW)
    hsum = racc[0] + racc[1]
    hacc4 = hsum[:NCHUNK, :ND]
    exs = hsum[NCHUNK, :ND, :EXW]

    return _finish_call(
        hacc4, exs, degv, p2, mix, p['W_O'], p['wl_w'],
        p['wl_b'].reshape(1, D),
        p['ffn_ln_g'].reshape(1, D), p['ffn_ln_b'].reshape(1, D),
        p['ffn_w1'], p['ffn_b1'].reshape(1, 4 * D),
        p['ffn_w2'], p['ffn_b2'].reshape(1, D))
